# Initial kernel scaffold; baseline (speedup 1.0000x reference)
import jax
import jax.numpy as jnp
from jax.experimental import pallas as pl


def kernel(x, edge_attr, edge_index, params):
    raise NotImplementedError("write your pallas kernel here")


# speedup vs baseline: 1.7606x; 1.7606x over previous
import functools

import jax
import jax.numpy as jnp
from jax import lax
from jax.experimental import pallas as pl
from jax.experimental.pallas import tpu as pltpu
from jax.experimental.pallas import tpu_sc as plsc

F32 = jnp.float32
HID = 512
_INTERPRET = False


def _ln(x, g, b, eps=1e-5):
    m = jnp.mean(x, axis=-1, keepdims=True)
    xc = x - m
    v = jnp.mean(xc * xc, axis=-1, keepdims=True)
    return xc * lax.rsqrt(v + eps) * g + b


def _embed_body(x_ref, w1_ref, b1_ref, g_ref, gb_ref, w2_ref, b2_ref, o_ref):
    t = jnp.dot(x_ref[...], w1_ref[...], preferred_element_type=F32) + b1_ref[...]
    t = jnp.maximum(t, 0.0)
    t = _ln(t, g_ref[...], gb_ref[...])
    o_ref[...] = jnp.dot(t, w2_ref[...], preferred_element_type=F32) + b2_ref[...]


def _embed(x, w1t, b1, g, gb, w2t, b2, bm):
    m, din = x.shape
    _rep = lambda i: (0, 0)
    return pl.pallas_call(
        _embed_body,
        grid=(m // bm,),
        in_specs=[
            pl.BlockSpec((bm, din), lambda i: (i, 0)),
            pl.BlockSpec((din, HID), _rep),
            pl.BlockSpec((1, HID), _rep),
            pl.BlockSpec((1, HID), _rep),
            pl.BlockSpec((1, HID), _rep),
            pl.BlockSpec((HID, HID), _rep),
            pl.BlockSpec((1, HID), _rep),
        ],
        out_specs=pl.BlockSpec((bm, HID), lambda i: (i, 0)),
        out_shape=jax.ShapeDtypeStruct((m, HID), F32),
        interpret=_INTERPRET,
    )(x, w1t, b1, g, gb, w2t, b2)


def _nodeproj_body(h_ref, w_ref, b_ref, a1_ref, ts_ref, td_ref):
    p = jnp.dot(h_ref[...], w_ref[...], preferred_element_type=F32) + b_ref[...]
    a1_ref[...] = p[:, 0:512]
    ts_ref[:, 0:512] = p[:, 512:1024]
    ts_ref[:, 512:1024] = p[:, 1536:2048]
    ts_ref[:, 1024:1536] = p[:, 2048:2560]
    td_ref[:, 0:512] = p[:, 1024:1536]
    td_ref[:, 512:1024] = p[:, 1536:2048]
    td_ref[:, 1024:1536] = p[:, 2048:2560]


def _nodeproj(h, wcat, bcat, bm):
    n = h.shape[0]
    _rep = lambda i: (0, 0)
    return pl.pallas_call(
        _nodeproj_body,
        grid=(n // bm,),
        in_specs=[
            pl.BlockSpec((bm, HID), lambda i: (i, 0)),
            pl.BlockSpec((HID, 2560), _rep),
            pl.BlockSpec((1, 2560), _rep),
        ],
        out_specs=[
            pl.BlockSpec((bm, HID), lambda i: (i, 0)),
            pl.BlockSpec((bm, 1536), lambda i: (i, 0)),
            pl.BlockSpec((bm, 1536), lambda i: (i, 0)),
        ],
        out_shape=[
            jax.ShapeDtypeStruct((n, HID), F32),
            jax.ShapeDtypeStruct((n, 1536), F32),
            jax.ShapeDtypeStruct((n, 1536), F32),
        ],
        interpret=_INTERPRET,
    )(h, wcat, bcat)


def _edgelayer_body(e_ref, gs_ref, gd_ref, w_ref, b_ref, g_ref, gb_ref,
                    enew_ref, mf_ref, mb_ref):
    e = e_ref[...]
    b1h = jnp.dot(e, w_ref[...], preferred_element_type=F32) + b_ref[...]
    gs = gs_ref[...]
    gd = gd_ref[...]
    g = g_ref[...]
    gb = gb_ref[...]
    efw = jnp.maximum(b1h + gs[:, 512:1024] + gd[:, 1024:1536], 0.0)
    ebw = jnp.maximum(b1h + gd[:, 512:1024] + gs[:, 1024:1536], 0.0)
    efw = e + _ln(efw, g, gb)
    ebw = e + _ln(ebw, g, gb)
    sf = jax.nn.sigmoid(efw)
    sb = jax.nn.sigmoid(ebw)
    mf_ref[...] = gs[:, 0:512] * sf / (jnp.sum(sf, axis=1, keepdims=True) + 1e-6)
    mb_ref[...] = gd[:, 0:512] * sb / (jnp.sum(sb, axis=1, keepdims=True) + 1e-6)
    enew_ref[...] = efw


def _edgelayer(e, gs, gd, wb1t, bb1, g, gb, be):
    m = e.shape[0]
    _rep = lambda i: (0, 0)
    blk = lambda i: (i, 0)
    return pl.pallas_call(
        _edgelayer_body,
        grid=(m // be,),
        in_specs=[
            pl.BlockSpec((be, HID), blk),
            pl.BlockSpec((be, 1536), blk),
            pl.BlockSpec((be, 1536), blk),
            pl.BlockSpec((HID, HID), _rep),
            pl.BlockSpec((1, HID), _rep),
            pl.BlockSpec((1, HID), _rep),
            pl.BlockSpec((1, HID), _rep),
        ],
        out_specs=[
            pl.BlockSpec((be, HID), blk),
            pl.BlockSpec((be, HID), blk),
            pl.BlockSpec((be, HID), blk),
        ],
        out_shape=[
            jax.ShapeDtypeStruct((m, HID), F32),
            jax.ShapeDtypeStruct((m, HID), F32),
            jax.ShapeDtypeStruct((m, HID), F32),
        ],
        interpret=_INTERPRET,
    )(e, gs, gd, wb1t, bb1, g, gb)


def _nodeupd_body(h_ref, a1_ref, hf_ref, hb_ref, g_ref, gb_ref, o_ref):
    t = jnp.maximum(a1_ref[...] + hf_ref[...] + hb_ref[...], 0.0)
    o_ref[...] = h_ref[...] + _ln(t, g_ref[...], gb_ref[...])


def _nodeupd(h, a1h, hf, hb, g, gb, bm):
    n = h.shape[0]
    _rep = lambda i: (0, 0)
    blk = lambda i: (i, 0)
    return pl.pallas_call(
        _nodeupd_body,
        grid=(n // bm,),
        in_specs=[pl.BlockSpec((bm, HID), blk)] * 4
        + [pl.BlockSpec((1, HID), _rep)] * 2,
        out_specs=pl.BlockSpec((bm, HID), blk),
        out_shape=jax.ShapeDtypeStruct((n, HID), F32),
        interpret=_INTERPRET,
    )(h, a1h, hf, hb, g, gb)


def _scoreproj_body(h_ref, w_ref, ha_ref, hb_ref):
    p = jnp.dot(h_ref[...], w_ref[...], preferred_element_type=F32)
    ha_ref[...] = p[:, 0:512]
    hb_ref[...] = p[:, 512:1024]


def _scoreproj(h, wab, bm):
    n = h.shape[0]
    return pl.pallas_call(
        _scoreproj_body,
        grid=(n // bm,),
        in_specs=[
            pl.BlockSpec((bm, HID), lambda i: (i, 0)),
            pl.BlockSpec((HID, 1024), lambda i: (0, 0)),
        ],
        out_specs=[
            pl.BlockSpec((bm, HID), lambda i: (i, 0)),
            pl.BlockSpec((bm, HID), lambda i: (i, 0)),
        ],
        out_shape=[
            jax.ShapeDtypeStruct((n, HID), F32),
            jax.ShapeDtypeStruct((n, HID), F32),
        ],
        interpret=_INTERPRET,
    )(h, wab)


def _scorefinal_body(ha_ref, hb_ref, e_ref, wc_ref, b1_ref, w2_ref, b2_ref,
                     o_ref):
    t = jnp.dot(e_ref[...], wc_ref[...], preferred_element_type=F32)
    t = jnp.maximum(t + ha_ref[...] + hb_ref[...] + b1_ref[...], 0.0)
    o_ref[...] = jnp.dot(t, w2_ref[...], preferred_element_type=F32) + b2_ref[...]


def _scorefinal(ha, hb, e, wct, b1, w2p, b2p, be):
    m = e.shape[0]
    _rep = lambda i: (0, 0)
    blk = lambda i: (i, 0)
    return pl.pallas_call(
        _scorefinal_body,
        grid=(m // be,),
        in_specs=[
            pl.BlockSpec((be, HID), blk),
            pl.BlockSpec((be, HID), blk),
            pl.BlockSpec((be, HID), blk),
            pl.BlockSpec((HID, HID), _rep),
            pl.BlockSpec((1, HID), _rep),
            pl.BlockSpec((HID, 128), _rep),
            pl.BlockSpec((1, 128), _rep),
        ],
        out_specs=pl.BlockSpec((be, 128), blk),
        out_shape=jax.ShapeDtypeStruct((m, 128), F32),
        interpret=_INTERPRET,
    )(ha, hb, e, wct, b1, w2p, b2p)


_NW = 32


def _gather_rows(table, idx):
    n, d = table.shape
    e = idx.shape[0]
    per = e // _NW
    c = 40
    nch = per // c
    mesh = plsc.VectorSubcoreMesh(core_axis_name="c", subcore_axis_name="s")

    @functools.partial(
        pl.kernel,
        mesh=mesh,
        out_type=jax.ShapeDtypeStruct((e, d), F32),
        scratch_types=[
            pltpu.VMEM((c,), jnp.int32),
            pltpu.VMEM((c, d), F32),
            pltpu.SemaphoreType.DMA,
        ],
    )
    def k(table_hbm, idx_hbm, out_hbm, idx_v, rows_v, sem):
        wid = lax.axis_index("s") * 2 + lax.axis_index("c")
        base = wid * per

        def body(j, carry):
            st = base + j * c
            pltpu.sync_copy(idx_hbm.at[pl.ds(st, c)], idx_v)
            pltpu.async_copy(table_hbm.at[idx_v], rows_v, sem).wait()
            pltpu.sync_copy(rows_v, out_hbm.at[pl.ds(st, c)])
            return carry

        lax.fori_loop(0, nch, body, 0)

    return k(table, idx)


def _segsum(msg, seg, n_out, zeros_fc):
    e, d = msg.shape
    fc = 128
    nfc_per_core = (d // fc) // 2
    per = e // 16
    c = 80
    nch = per // c
    rpt = n_out // 16
    mesh = plsc.VectorSubcoreMesh(core_axis_name="c", subcore_axis_name="s")

    @functools.partial(
        pl.kernel,
        mesh=mesh,
        out_type=jax.ShapeDtypeStruct((n_out, d), F32),
        scratch_types=[
            pltpu.VMEM((c,), jnp.int32),
            pltpu.VMEM((c, fc), F32),
            pltpu.VMEM_SHARED((n_out, fc), F32),
            pltpu.SemaphoreType.DMA,
        ],
    )
    def k(msg_hbm, seg_hbm, z_hbm, out_hbm, idx_v, m_v, acc, sem):
        cid = lax.axis_index("c")
        sid = lax.axis_index("s")

        def do_fc(fci, carry):
            col = (cid * nfc_per_core + fci) * fc
            pltpu.sync_copy(z_hbm.at[pl.ds(sid * rpt, rpt)],
                            acc.at[pl.ds(sid * rpt, rpt)])
            plsc.subcore_barrier()

            def body(j, cc):
                st = sid * per + j * c
                pltpu.sync_copy(seg_hbm.at[pl.ds(st, c)], idx_v)
                pltpu.sync_copy(msg_hbm.at[pl.ds(st, c), pl.ds(col, fc)], m_v)
                pltpu.sync_copy(m_v, acc.at[idx_v], add=True)
                return cc

            lax.fori_loop(0, nch, body, 0)
            plsc.subcore_barrier()
            pltpu.sync_copy(acc.at[pl.ds(sid * rpt, rpt)],
                            out_hbm.at[pl.ds(sid * rpt, rpt), pl.ds(col, fc)])
            plsc.subcore_barrier()
            return carry

        lax.fori_loop(0, nfc_per_core, do_fc, 0)

    return k(msg, seg, zeros_fc)


def kernel(x, edge_attr, edge_index, params):
    src = edge_index[0]
    dst = edge_index[1]
    n = x.shape[0]
    p = params

    def t(wb):
        return wb[0].T

    def b2d(wb):
        return wb[1].reshape(1, -1)

    h = _embed(x, t(p['W11']), b2d(p['W11']),
               p['ln1'][0].reshape(1, -1), p['ln1'][1].reshape(1, -1),
               t(p['W12']), b2d(p['W12']), bm=400)
    e = _embed(edge_attr, t(p['W21']), b2d(p['W21']),
               p['ln2'][0].reshape(1, -1), p['ln2'][1].reshape(1, -1),
               t(p['W22']), b2d(p['W22']), bm=640)

    npad = ((n + 127) // 128) * 128
    zeros_fc = jnp.zeros((npad, 128), F32)

    for lp in p['layers']:
        wcat = jnp.concatenate(
            [t(lp['A1']), t(lp['A2']), t(lp['A3']), t(lp['B2']), t(lp['B3'])],
            axis=1)
        bcat = jnp.concatenate(
            [b2d(lp['A1']), b2d(lp['A2']), b2d(lp['A3']), b2d(lp['B2']),
             b2d(lp['B3'])], axis=1)
        a1h, tsrc, tdst = _nodeproj(h, wcat, bcat, bm=400)
        gs = _gather_rows(tsrc, src)
        gd = _gather_rows(tdst, dst)
        e, mf, mb = _edgelayer(
            e, gs, gd, t(lp['B1']), b2d(lp['B1']),
            lp['ln_e'][0].reshape(1, -1), lp['ln_e'][1].reshape(1, -1),
            be=640)
        hf = _segsum(mf, dst, npad, zeros_fc)[:n]
        hb = _segsum(mb, src, npad, zeros_fc)[:n]
        h = _nodeupd(h, a1h, hf, hb,
                     lp['ln_h'][0].reshape(1, -1),
                     lp['ln_h'][1].reshape(1, -1), bm=400)

    w1, b1 = p['scorer1']
    wab = jnp.concatenate([w1[:, 0:512].T, w1[:, 512:1024].T], axis=1)
    ha, hb2 = _scoreproj(h, wab, bm=400)
    has = _gather_rows(ha, src)
    hbd = _gather_rows(hb2, dst)
    w2, b2 = p['scorer2']
    w2p = jnp.zeros((HID, 128), F32).at[:, 0].set(w2[0])
    b2p = jnp.broadcast_to(b2.reshape(1, 1), (1, 128)).astype(F32)
    sc = _scorefinal(has, hbd, e, w1[:, 1024:1536].T, b1.reshape(1, -1),
                     w2p, b2p, be=640)
    return sc[:, 0:1]


# speedup vs baseline: 2.5979x; 1.4755x over previous
import functools

import jax
import jax.numpy as jnp
from jax import lax
from jax.experimental import pallas as pl
from jax.experimental.pallas import tpu as pltpu
from jax.experimental.pallas import tpu_sc as plsc

F32 = jnp.float32
HID = 512
_INTERPRET = False


def _ln(x, g, b, eps=1e-5):
    m = jnp.mean(x, axis=-1, keepdims=True)
    xc = x - m
    v = jnp.mean(xc * xc, axis=-1, keepdims=True)
    return xc * lax.rsqrt(v + eps) * g + b


_HI_MASK = -65536


def _pack_bf16(a, b):
    ai = lax.bitcast_convert_type(a, jnp.int32)
    bi = lax.bitcast_convert_type(b, jnp.int32)
    lo = ((ai + 0x8000) >> 16) & 0xFFFF
    hi = (bi + 0x8000) & _HI_MASK
    return hi | lo


def _unpack_bf16(w):
    lo = lax.bitcast_convert_type(w << 16, F32)
    hi = lax.bitcast_convert_type(w & _HI_MASK, F32)
    return lo, hi


def _embed_body(x_ref, w1_ref, b1_ref, g_ref, gb_ref, w2_ref, b2_ref, o_ref):
    t = jnp.dot(x_ref[...], w1_ref[...], preferred_element_type=F32) + b1_ref[...]
    t = jnp.maximum(t, 0.0)
    t = _ln(t, g_ref[...], gb_ref[...])
    o_ref[...] = jnp.dot(t, w2_ref[...], preferred_element_type=F32) + b2_ref[...]


def _embed(x, w1t, b1, g, gb, w2t, b2, bm):
    m, din = x.shape
    _rep = lambda i: (0, 0)
    return pl.pallas_call(
        _embed_body,
        grid=(m // bm,),
        in_specs=[
            pl.BlockSpec((bm, din), lambda i: (i, 0)),
            pl.BlockSpec((din, HID), _rep),
            pl.BlockSpec((1, HID), _rep),
            pl.BlockSpec((1, HID), _rep),
            pl.BlockSpec((1, HID), _rep),
            pl.BlockSpec((HID, HID), _rep),
            pl.BlockSpec((1, HID), _rep),
        ],
        out_specs=pl.BlockSpec((bm, HID), lambda i: (i, 0)),
        out_shape=jax.ShapeDtypeStruct((m, HID), F32),
        interpret=_INTERPRET,
    )(x, w1t, b1, g, gb, w2t, b2)


def _nodeproj_body(h_ref, w_ref, b_ref, a1_ref, ts_ref, td_ref):
    p = jnp.dot(h_ref[...], w_ref[...], preferred_element_type=F32) + b_ref[...]
    a1_ref[...] = p[:, 0:512]
    ts = jnp.concatenate([p[:, 512:1024], p[:, 1536:2048], p[:, 2048:2560]],
                         axis=1)
    td = jnp.concatenate([p[:, 1024:1536], p[:, 1536:2048], p[:, 2048:2560]],
                         axis=1)
    ts_ref[...] = _pack_bf16(ts[:, 0:768], ts[:, 768:1536])
    td_ref[...] = _pack_bf16(td[:, 0:768], td[:, 768:1536])


def _nodeproj(h, wcat, bcat, bm):
    n = h.shape[0]
    _rep = lambda i: (0, 0)
    return pl.pallas_call(
        _nodeproj_body,
        grid=(n // bm,),
        in_specs=[
            pl.BlockSpec((bm, HID), lambda i: (i, 0)),
            pl.BlockSpec((HID, 2560), _rep),
            pl.BlockSpec((1, 2560), _rep),
        ],
        out_specs=[
            pl.BlockSpec((bm, HID), lambda i: (i, 0)),
            pl.BlockSpec((bm, 768), lambda i: (i, 0)),
            pl.BlockSpec((bm, 768), lambda i: (i, 0)),
        ],
        out_shape=[
            jax.ShapeDtypeStruct((n, HID), F32),
            jax.ShapeDtypeStruct((n, 768), jnp.int32),
            jax.ShapeDtypeStruct((n, 768), jnp.int32),
        ],
        interpret=_INTERPRET,
    )(h, wcat, bcat)


def _edgelayer_body(e_ref, gs_ref, gd_ref, w_ref, b_ref, g_ref, gb_ref,
                    enew_ref, mf_ref, mb_ref):
    e = e_ref[...]
    b1h = jnp.dot(e, w_ref[...], preferred_element_type=F32) + b_ref[...]
    gslo, gshi = _unpack_bf16(gs_ref[...])
    gdlo, gdhi = _unpack_bf16(gd_ref[...])
    a2s = gslo[:, 0:512]
    b2s = jnp.concatenate([gslo[:, 512:768], gshi[:, 0:256]], axis=1)
    b3s = gshi[:, 256:768]
    a3d = gdlo[:, 0:512]
    b2d = jnp.concatenate([gdlo[:, 512:768], gdhi[:, 0:256]], axis=1)
    b3d = gdhi[:, 256:768]
    g = g_ref[...]
    gb = gb_ref[...]
    efw = jnp.maximum(b1h + b2s + b3d, 0.0)
    ebw = jnp.maximum(b1h + b2d + b3s, 0.0)
    efw = e + _ln(efw, g, gb)
    ebw = e + _ln(ebw, g, gb)
    sf = jax.nn.sigmoid(efw)
    sb = jax.nn.sigmoid(ebw)
    mf_ref[...] = a2s * sf / (jnp.sum(sf, axis=1, keepdims=True) + 1e-6)
    mb_ref[...] = a3d * sb / (jnp.sum(sb, axis=1, keepdims=True) + 1e-6)
    enew_ref[...] = efw


def _edgelayer(e, gs, gd, wb1t, bb1, g, gb, be):
    m = e.shape[0]
    _rep = lambda i: (0, 0)
    blk = lambda i: (i, 0)
    return pl.pallas_call(
        _edgelayer_body,
        grid=(m // be,),
        in_specs=[
            pl.BlockSpec((be, HID), blk),
            pl.BlockSpec((be, 768), blk),
            pl.BlockSpec((be, 768), blk),
            pl.BlockSpec((HID, HID), _rep),
            pl.BlockSpec((1, HID), _rep),
            pl.BlockSpec((1, HID), _rep),
            pl.BlockSpec((1, HID), _rep),
        ],
        out_specs=[
            pl.BlockSpec((be, HID), blk),
            pl.BlockSpec((be, HID), blk),
            pl.BlockSpec((be, HID), blk),
        ],
        out_shape=[
            jax.ShapeDtypeStruct((m, HID), F32),
            jax.ShapeDtypeStruct((m, HID), F32),
            jax.ShapeDtypeStruct((m, HID), F32),
        ],
        interpret=_INTERPRET,
    )(e, gs, gd, wb1t, bb1, g, gb)


def _nodeupd_body(h_ref, a1_ref, hs_ref, g_ref, gb_ref, o_ref):
    t = jnp.maximum(a1_ref[...] + hs_ref[...], 0.0)
    o_ref[...] = h_ref[...] + _ln(t, g_ref[...], gb_ref[...])


def _nodeupd(h, a1h, hs, g, gb, bm):
    n = h.shape[0]
    _rep = lambda i: (0, 0)
    blk = lambda i: (i, 0)
    return pl.pallas_call(
        _nodeupd_body,
        grid=(n // bm,),
        in_specs=[pl.BlockSpec((bm, HID), blk)] * 3
        + [pl.BlockSpec((1, HID), _rep)] * 2,
        out_specs=pl.BlockSpec((bm, HID), blk),
        out_shape=jax.ShapeDtypeStruct((n, HID), F32),
        interpret=_INTERPRET,
    )(h, a1h, hs, g, gb)


def _scoreproj_body(h_ref, w_ref, ha_ref, hb_ref):
    p = jnp.dot(h_ref[...], w_ref[...], preferred_element_type=F32)
    ha_ref[...] = _pack_bf16(p[:, 0:256], p[:, 256:512])
    hb_ref[...] = _pack_bf16(p[:, 512:768], p[:, 768:1024])


def _scoreproj(h, wab, bm):
    n = h.shape[0]
    return pl.pallas_call(
        _scoreproj_body,
        grid=(n // bm,),
        in_specs=[
            pl.BlockSpec((bm, HID), lambda i: (i, 0)),
            pl.BlockSpec((HID, 1024), lambda i: (0, 0)),
        ],
        out_specs=[
            pl.BlockSpec((bm, 256), lambda i: (i, 0)),
            pl.BlockSpec((bm, 256), lambda i: (i, 0)),
        ],
        out_shape=[
            jax.ShapeDtypeStruct((n, 256), jnp.int32),
            jax.ShapeDtypeStruct((n, 256), jnp.int32),
        ],
        interpret=_INTERPRET,
    )(h, wab)


def _scorefinal_body(ha_ref, hb_ref, e_ref, wc_ref, b1_ref, w2_ref, b2_ref,
                     o_ref):
    halo, hahi = _unpack_bf16(ha_ref[...])
    hblo, hbhi = _unpack_bf16(hb_ref[...])
    ha = jnp.concatenate([halo, hahi], axis=1)
    hb = jnp.concatenate([hblo, hbhi], axis=1)
    t = jnp.dot(e_ref[...], wc_ref[...], preferred_element_type=F32)
    t = jnp.maximum(t + ha + hb + b1_ref[...], 0.0)
    o_ref[...] = jnp.dot(t, w2_ref[...], preferred_element_type=F32) + b2_ref[...]


def _scorefinal(ha, hb, e, wct, b1, w2p, b2p, be):
    m = e.shape[0]
    _rep = lambda i: (0, 0)
    blk = lambda i: (i, 0)
    return pl.pallas_call(
        _scorefinal_body,
        grid=(m // be,),
        in_specs=[
            pl.BlockSpec((be, 256), blk),
            pl.BlockSpec((be, 256), blk),
            pl.BlockSpec((be, HID), blk),
            pl.BlockSpec((HID, HID), _rep),
            pl.BlockSpec((1, HID), _rep),
            pl.BlockSpec((HID, 128), _rep),
            pl.BlockSpec((1, 128), _rep),
        ],
        out_specs=pl.BlockSpec((be, 128), blk),
        out_shape=jax.ShapeDtypeStruct((m, 128), F32),
        interpret=_INTERPRET,
    )(ha, hb, e, wct, b1, w2p, b2p)


_NW = 32


def _gather_rows(table, idx):
    n, d = table.shape
    dt = table.dtype
    e = idx.shape[0]
    per = e // _NW
    c = 40
    nch = per // c
    idx3 = idx.reshape(_NW, nch, c)
    mesh = plsc.VectorSubcoreMesh(core_axis_name="c", subcore_axis_name="s")

    @functools.partial(
        pl.kernel,
        mesh=mesh,
        out_type=jax.ShapeDtypeStruct((e, d), dt),
        scratch_types=[
            pltpu.VMEM((nch, c), jnp.int32),
            pltpu.VMEM((c, d), dt),
            pltpu.SemaphoreType.DMA,
        ],
    )
    def k(table_hbm, idx_hbm, out_hbm, idx_v, rows_v, sem):
        wid = lax.axis_index("s") * 2 + lax.axis_index("c")
        base = wid * per
        pltpu.sync_copy(idx_hbm.at[wid], idx_v)

        def body(j, carry):
            pltpu.async_copy(table_hbm.at[idx_v.at[j]], rows_v, sem).wait()
            pltpu.sync_copy(rows_v, out_hbm.at[pl.ds(base + j * c, c)])
            return carry

        lax.fori_loop(0, nch, body, 0)

    return k(table, idx3)


def _segsum2(msg_f, seg_f3, msg_b, seg_b3, n_out, zeros_fc):
    e, d = msg_f.shape
    fc = 128
    nfc_per_core = (d // fc) // 2
    per = e // 16
    c = 80
    nch = per // c
    rpt = n_out // 16
    mesh = plsc.VectorSubcoreMesh(core_axis_name="c", subcore_axis_name="s")

    @functools.partial(
        pl.kernel,
        mesh=mesh,
        out_type=jax.ShapeDtypeStruct((n_out, d), F32),
        scratch_types=[
            pltpu.VMEM((nch, c), jnp.int32),
            pltpu.VMEM((nch, c), jnp.int32),
            pltpu.VMEM((c, fc), F32),
            pltpu.VMEM_SHARED((n_out, fc), F32),
            pltpu.SemaphoreType.DMA,
        ],
    )
    def k(mf_hbm, sf_hbm, mb_hbm, sb_hbm, z_hbm, out_hbm,
          idxf_v, idxb_v, m_v, acc, sem):
        cid = lax.axis_index("c")
        sid = lax.axis_index("s")
        pltpu.sync_copy(sf_hbm.at[sid], idxf_v)
        pltpu.sync_copy(sb_hbm.at[sid], idxb_v)

        def do_fc(fci, carry):
            col = (cid * nfc_per_core + fci) * fc
            pltpu.sync_copy(z_hbm.at[pl.ds(sid * rpt, rpt)],
                            acc.at[pl.ds(sid * rpt, rpt)])
            plsc.subcore_barrier()

            def body(j, cc):
                st = sid * per + j * c
                pltpu.sync_copy(mf_hbm.at[pl.ds(st, c), pl.ds(col, fc)], m_v)
                pltpu.sync_copy(m_v, acc.at[idxf_v.at[j]], add=True)
                pltpu.sync_copy(mb_hbm.at[pl.ds(st, c), pl.ds(col, fc)], m_v)
                pltpu.sync_copy(m_v, acc.at[idxb_v.at[j]], add=True)
                return cc

            lax.fori_loop(0, nch, body, 0)
            plsc.subcore_barrier()
            pltpu.sync_copy(acc.at[pl.ds(sid * rpt, rpt)],
                            out_hbm.at[pl.ds(sid * rpt, rpt), pl.ds(col, fc)])
            plsc.subcore_barrier()
            return carry

        lax.fori_loop(0, nfc_per_core, do_fc, 0)

    return k(msg_f, seg_f3, msg_b, seg_b3, zeros_fc)


def kernel(x, edge_attr, edge_index, params):
    src = edge_index[0]
    dst = edge_index[1]
    n = x.shape[0]
    p = params

    def t(wb):
        return wb[0].T

    def b2d(wb):
        return wb[1].reshape(1, -1)

    h = _embed(x, t(p['W11']), b2d(p['W11']),
               p['ln1'][0].reshape(1, -1), p['ln1'][1].reshape(1, -1),
               t(p['W12']), b2d(p['W12']), bm=400)
    e = _embed(edge_attr, t(p['W21']), b2d(p['W21']),
               p['ln2'][0].reshape(1, -1), p['ln2'][1].reshape(1, -1),
               t(p['W22']), b2d(p['W22']), bm=640)

    npad = ((n + 127) // 128) * 128
    zeros_fc = jnp.zeros((npad, 128), F32)
    ne = src.shape[0]
    src3 = src.reshape(16, (ne // 16) // 80, 80)
    dst3 = dst.reshape(16, (ne // 16) // 80, 80)

    for lp in p['layers']:
        wcat = jnp.concatenate(
            [t(lp['A1']), t(lp['A2']), t(lp['A3']), t(lp['B2']), t(lp['B3'])],
            axis=1)
        bcat = jnp.concatenate(
            [b2d(lp['A1']), b2d(lp['A2']), b2d(lp['A3']), b2d(lp['B2']),
             b2d(lp['B3'])], axis=1)
        a1h, tsrc, tdst = _nodeproj(h, wcat, bcat, bm=400)
        gs = _gather_rows(tsrc, src)
        gd = _gather_rows(tdst, dst)
        e, mf, mb = _edgelayer(
            e, gs, gd, t(lp['B1']), b2d(lp['B1']),
            lp['ln_e'][0].reshape(1, -1), lp['ln_e'][1].reshape(1, -1),
            be=640)
        hs = _segsum2(mf, dst3, mb, src3, npad, zeros_fc)[:n]
        h = _nodeupd(h, a1h, hs,
                     lp['ln_h'][0].reshape(1, -1),
                     lp['ln_h'][1].reshape(1, -1), bm=400)

    w1, b1 = p['scorer1']
    wab = jnp.concatenate([w1[:, 0:512].T, w1[:, 512:1024].T], axis=1)
    ha, hb2 = _scoreproj(h, wab, bm=400)
    has = _gather_rows(ha, src)
    hbd = _gather_rows(hb2, dst)
    w2, b2 = p['scorer2']
    w2p = jnp.zeros((HID, 128), F32).at[:, 0].set(w2[0])
    b2p = jnp.broadcast_to(b2.reshape(1, 1), (1, 128)).astype(F32)
    sc = _scorefinal(has, hbd, e, w1[:, 1024:1536].T, b1.reshape(1, -1),
                     w2p, b2p, be=640)
    return sc[:, 0:1]


# speedup vs baseline: 2.7903x; 1.0741x over previous
import functools

import jax
import jax.numpy as jnp
from jax import lax
from jax.experimental import pallas as pl
from jax.experimental.pallas import tpu as pltpu
from jax.experimental.pallas import tpu_sc as plsc

F32 = jnp.float32
BF16 = jnp.bfloat16
HID = 512
_INTERPRET = False


def _ln(x, g, b, eps=1e-5):
    m = jnp.mean(x, axis=-1, keepdims=True)
    xc = x - m
    v = jnp.mean(xc * xc, axis=-1, keepdims=True)
    return xc * lax.rsqrt(v + eps) * g + b


_HI_MASK = -65536


def _pack_bf16(a, b):
    ai = lax.bitcast_convert_type(a, jnp.int32)
    bi = lax.bitcast_convert_type(b, jnp.int32)
    lo = ((ai + 0x8000) >> 16) & 0xFFFF
    hi = (bi + 0x8000) & _HI_MASK
    return hi | lo


def _unpack_bf16(w):
    lo = lax.bitcast_convert_type(w << 16, F32)
    hi = lax.bitcast_convert_type(w & _HI_MASK, F32)
    return lo, hi


def _embed_body(x_ref, w1_ref, b1_ref, g_ref, gb_ref, w2_ref, b2_ref, o_ref):
    t = jnp.dot(x_ref[...], w1_ref[...], preferred_element_type=F32) + b1_ref[...]
    t = jnp.maximum(t, 0.0)
    t = _ln(t, g_ref[...], gb_ref[...])
    o_ref[...] = jnp.dot(t, w2_ref[...], preferred_element_type=F32) + b2_ref[...]


def _embed(x, w1t, b1, g, gb, w2t, b2, bm):
    m, din = x.shape
    _rep = lambda i: (0, 0)
    return pl.pallas_call(
        _embed_body,
        grid=(m // bm,),
        in_specs=[
            pl.BlockSpec((bm, din), lambda i: (i, 0)),
            pl.BlockSpec((din, HID), _rep),
            pl.BlockSpec((1, HID), _rep),
            pl.BlockSpec((1, HID), _rep),
            pl.BlockSpec((1, HID), _rep),
            pl.BlockSpec((HID, HID), _rep),
            pl.BlockSpec((1, HID), _rep),
        ],
        out_specs=pl.BlockSpec((bm, HID), lambda i: (i, 0)),
        out_shape=jax.ShapeDtypeStruct((m, HID), F32),
        interpret=_INTERPRET,
    )(x, w1t, b1, g, gb, w2t, b2)


def _nodeproj_body(h_ref, w1_ref, b1_ref, wt_ref, bt_ref, a1_ref, ts_ref,
                   td_ref):
    h = h_ref[...]
    a1_ref[...] = jnp.dot(h, w1_ref[...],
                          preferred_element_type=F32) + b1_ref[...]
    p = jnp.dot(h.astype(BF16), wt_ref[...],
                preferred_element_type=F32) + bt_ref[...]
    ts = jnp.concatenate([p[:, 0:512], p[:, 1024:1536], p[:, 1536:2048]],
                         axis=1)
    td = jnp.concatenate([p[:, 512:1024], p[:, 1024:1536], p[:, 1536:2048]],
                         axis=1)
    ts_ref[...] = _pack_bf16(ts[:, 0:768], ts[:, 768:1536])
    td_ref[...] = _pack_bf16(td[:, 0:768], td[:, 768:1536])


def _nodeproj(h, w1, b1, wt, bt, bm):
    n = h.shape[0]
    _rep = lambda i: (0, 0)
    return pl.pallas_call(
        _nodeproj_body,
        grid=(n // bm,),
        in_specs=[
            pl.BlockSpec((bm, HID), lambda i: (i, 0)),
            pl.BlockSpec((HID, 512), _rep),
            pl.BlockSpec((1, 512), _rep),
            pl.BlockSpec((HID, 2048), _rep),
            pl.BlockSpec((1, 2048), _rep),
        ],
        out_specs=[
            pl.BlockSpec((bm, HID), lambda i: (i, 0)),
            pl.BlockSpec((bm, 768), lambda i: (i, 0)),
            pl.BlockSpec((bm, 768), lambda i: (i, 0)),
        ],
        out_shape=[
            jax.ShapeDtypeStruct((n, HID), F32),
            jax.ShapeDtypeStruct((n, 768), jnp.int32),
            jax.ShapeDtypeStruct((n, 768), jnp.int32),
        ],
        interpret=_INTERPRET,
    )(h, w1, b1, wt, bt)


def _edgelayer_body(e_ref, gs_ref, gd_ref, w_ref, b_ref, g_ref, gb_ref,
                    enew_ref, mf_ref, mb_ref):
    e = e_ref[...]
    b1h = jnp.dot(e, w_ref[...], preferred_element_type=F32) + b_ref[...]
    gslo, gshi = _unpack_bf16(gs_ref[...])
    gdlo, gdhi = _unpack_bf16(gd_ref[...])
    a2s = gslo[:, 0:512]
    b2s = jnp.concatenate([gslo[:, 512:768], gshi[:, 0:256]], axis=1)
    b3s = gshi[:, 256:768]
    a3d = gdlo[:, 0:512]
    b2d = jnp.concatenate([gdlo[:, 512:768], gdhi[:, 0:256]], axis=1)
    b3d = gdhi[:, 256:768]
    g = g_ref[...]
    gb = gb_ref[...]
    efw = jnp.maximum(b1h + b2s + b3d, 0.0)
    ebw = jnp.maximum(b1h + b2d + b3s, 0.0)
    efw = e + _ln(efw, g, gb)
    ebw = e + _ln(ebw, g, gb)
    sf = jax.nn.sigmoid(efw)
    sb = jax.nn.sigmoid(ebw)
    mf_ref[...] = a2s * sf / (jnp.sum(sf, axis=1, keepdims=True) + 1e-6)
    mb_ref[...] = a3d * sb / (jnp.sum(sb, axis=1, keepdims=True) + 1e-6)
    enew_ref[...] = efw


def _edgelayer(e, gs, gd, wb1t, bb1, g, gb, be):
    m = e.shape[0]
    _rep = lambda i: (0, 0)
    blk = lambda i: (i, 0)
    return pl.pallas_call(
        _edgelayer_body,
        grid=(m // be,),
        in_specs=[
            pl.BlockSpec((be, HID), blk),
            pl.BlockSpec((be, 768), blk),
            pl.BlockSpec((be, 768), blk),
            pl.BlockSpec((HID, HID), _rep),
            pl.BlockSpec((1, HID), _rep),
            pl.BlockSpec((1, HID), _rep),
            pl.BlockSpec((1, HID), _rep),
        ],
        out_specs=[
            pl.BlockSpec((be, HID), blk),
            pl.BlockSpec((be, HID), blk),
            pl.BlockSpec((be, HID), blk),
        ],
        out_shape=[
            jax.ShapeDtypeStruct((m, HID), F32),
            jax.ShapeDtypeStruct((m, HID), F32),
            jax.ShapeDtypeStruct((m, HID), F32),
        ],
        interpret=_INTERPRET,
    )(e, gs, gd, wb1t, bb1, g, gb)


def _nodeupd_body(h_ref, a1_ref, hs_ref, g_ref, gb_ref, o_ref):
    t = jnp.maximum(a1_ref[...] + hs_ref[...], 0.0)
    o_ref[...] = h_ref[...] + _ln(t, g_ref[...], gb_ref[...])


def _nodeupd(h, a1h, hs, g, gb, bm):
    n = h.shape[0]
    _rep = lambda i: (0, 0)
    blk = lambda i: (i, 0)
    return pl.pallas_call(
        _nodeupd_body,
        grid=(n // bm,),
        in_specs=[pl.BlockSpec((bm, HID), blk)] * 3
        + [pl.BlockSpec((1, HID), _rep)] * 2,
        out_specs=pl.BlockSpec((bm, HID), blk),
        out_shape=jax.ShapeDtypeStruct((n, HID), F32),
        interpret=_INTERPRET,
    )(h, a1h, hs, g, gb)


def _scoreproj_body(h_ref, w_ref, ha_ref, hb_ref):
    p = jnp.dot(h_ref[...].astype(BF16), w_ref[...], preferred_element_type=F32)
    ha_ref[...] = _pack_bf16(p[:, 0:256], p[:, 256:512])
    hb_ref[...] = _pack_bf16(p[:, 512:768], p[:, 768:1024])


def _scoreproj(h, wab, bm):
    n = h.shape[0]
    return pl.pallas_call(
        _scoreproj_body,
        grid=(n // bm,),
        in_specs=[
            pl.BlockSpec((bm, HID), lambda i: (i, 0)),
            pl.BlockSpec((HID, 1024), lambda i: (0, 0)),
        ],
        out_specs=[
            pl.BlockSpec((bm, 256), lambda i: (i, 0)),
            pl.BlockSpec((bm, 256), lambda i: (i, 0)),
        ],
        out_shape=[
            jax.ShapeDtypeStruct((n, 256), jnp.int32),
            jax.ShapeDtypeStruct((n, 256), jnp.int32),
        ],
        interpret=_INTERPRET,
    )(h, wab)


def _scorefinal_body(ha_ref, hb_ref, e_ref, wc_ref, b1_ref, w2_ref, b2_ref,
                     o_ref):
    halo, hahi = _unpack_bf16(ha_ref[...])
    hblo, hbhi = _unpack_bf16(hb_ref[...])
    ha = jnp.concatenate([halo, hahi], axis=1)
    hb = jnp.concatenate([hblo, hbhi], axis=1)
    t = jnp.dot(e_ref[...], wc_ref[...], preferred_element_type=F32)
    t = jnp.maximum(t + ha + hb + b1_ref[...], 0.0)
    o_ref[...] = jnp.dot(t, w2_ref[...], preferred_element_type=F32) + b2_ref[...]


def _scorefinal(ha, hb, e, wct, b1, w2p, b2p, be):
    m = e.shape[0]
    _rep = lambda i: (0, 0)
    blk = lambda i: (i, 0)
    return pl.pallas_call(
        _scorefinal_body,
        grid=(m // be,),
        in_specs=[
            pl.BlockSpec((be, 256), blk),
            pl.BlockSpec((be, 256), blk),
            pl.BlockSpec((be, HID), blk),
            pl.BlockSpec((HID, HID), _rep),
            pl.BlockSpec((1, HID), _rep),
            pl.BlockSpec((HID, 128), _rep),
            pl.BlockSpec((1, 128), _rep),
        ],
        out_specs=pl.BlockSpec((be, 128), blk),
        out_shape=jax.ShapeDtypeStruct((m, 128), F32),
        interpret=_INTERPRET,
    )(ha, hb, e, wct, b1, w2p, b2p)


_NW = 32


def _gather_rows(table, idx):
    n, d = table.shape
    dt = table.dtype
    e = idx.shape[0]
    per = e // _NW
    c = 40
    nch = per // c
    assert nch % 2 == 1
    idx3 = idx.reshape(_NW, nch, c)
    mesh = plsc.VectorSubcoreMesh(core_axis_name="c", subcore_axis_name="s")

    @functools.partial(
        pl.kernel,
        mesh=mesh,
        out_type=jax.ShapeDtypeStruct((e, d), dt),
        scratch_types=[
            pltpu.VMEM((nch, c), jnp.int32),
            pltpu.VMEM((2, c, d), dt),
            pltpu.SemaphoreType.DMA,
            pltpu.SemaphoreType.DMA,
            pltpu.SemaphoreType.DMA,
            pltpu.SemaphoreType.DMA,
        ],
    )
    def k(table_hbm, idx_hbm, out_hbm, idx_v, rows_v, sg0, sg1, sw0, sw1):
        wid = lax.axis_index("s") * 2 + lax.axis_index("c")
        base = wid * per
        sg = (sg0, sg1)
        sw = (sw0, sw1)
        pltpu.sync_copy(idx_hbm.at[wid], idx_v)
        pltpu.async_copy(table_hbm.at[idx_v.at[0]], rows_v.at[0], sg[0])

        def outer(g, carry):
            for b in range(2):
                j = g * 2 + b
                nb = 1 - b

                @pl.when(j >= 1)
                def _():
                    pltpu.make_async_copy(
                        rows_v.at[nb], out_hbm.at[pl.ds(base, c)], sw[nb]
                    ).wait()

                pltpu.async_copy(table_hbm.at[idx_v.at[j + 1]],
                                 rows_v.at[nb], sg[nb])
                pltpu.make_async_copy(table_hbm.at[idx_v.at[j]],
                                      rows_v.at[b], sg[b]).wait()
                pltpu.async_copy(rows_v.at[b],
                                 out_hbm.at[pl.ds(base + j * c, c)], sw[b])
            return carry

        lax.fori_loop(0, nch // 2, outer, 0)
        pltpu.make_async_copy(table_hbm.at[idx_v.at[nch - 1]],
                              rows_v.at[0], sg[0]).wait()
        pltpu.sync_copy(rows_v.at[0], out_hbm.at[pl.ds(base + (nch - 1) * c, c)])
        pltpu.make_async_copy(rows_v.at[1], out_hbm.at[pl.ds(base, c)],
                              sw[1]).wait()

    return k(table, idx3)


def _segsum2(msg_f, seg_f3, msg_b, seg_b3, n_out, zeros_fc):
    e, d = msg_f.shape
    fc = 128
    nfc_per_core = (d // fc) // 2
    per = e // 16
    c = 80
    nch = per // c
    rpt = n_out // 16
    mesh = plsc.VectorSubcoreMesh(core_axis_name="c", subcore_axis_name="s")

    @functools.partial(
        pl.kernel,
        mesh=mesh,
        out_type=jax.ShapeDtypeStruct((n_out, d), F32),
        scratch_types=[
            pltpu.VMEM((nch, c), jnp.int32),
            pltpu.VMEM((nch, c), jnp.int32),
            pltpu.VMEM((c, fc), F32),
            pltpu.VMEM_SHARED((n_out, fc), F32),
            pltpu.SemaphoreType.DMA,
        ],
    )
    def k(mf_hbm, sf_hbm, mb_hbm, sb_hbm, z_hbm, out_hbm,
          idxf_v, idxb_v, m_v, acc, sem):
        cid = lax.axis_index("c")
        sid = lax.axis_index("s")
        pltpu.sync_copy(sf_hbm.at[sid], idxf_v)
        pltpu.sync_copy(sb_hbm.at[sid], idxb_v)

        def do_fc(fci, carry):
            col = (cid * nfc_per_core + fci) * fc
            pltpu.sync_copy(z_hbm.at[pl.ds(sid * rpt, rpt)],
                            acc.at[pl.ds(sid * rpt, rpt)])
            plsc.subcore_barrier()

            def body(j, cc):
                st = sid * per + j * c
                pltpu.sync_copy(mf_hbm.at[pl.ds(st, c), pl.ds(col, fc)], m_v)
                pltpu.sync_copy(m_v, acc.at[idxf_v.at[j]], add=True)
                pltpu.sync_copy(mb_hbm.at[pl.ds(st, c), pl.ds(col, fc)], m_v)
                pltpu.sync_copy(m_v, acc.at[idxb_v.at[j]], add=True)
                return cc

            lax.fori_loop(0, nch, body, 0)
            plsc.subcore_barrier()
            pltpu.sync_copy(acc.at[pl.ds(sid * rpt, rpt)],
                            out_hbm.at[pl.ds(sid * rpt, rpt), pl.ds(col, fc)])
            plsc.subcore_barrier()
            return carry

        lax.fori_loop(0, nfc_per_core, do_fc, 0)

    return k(msg_f, seg_f3, msg_b, seg_b3, zeros_fc)


def kernel(x, edge_attr, edge_index, params):
    src = edge_index[0]
    dst = edge_index[1]
    n = x.shape[0]
    p = params

    def t(wb):
        return wb[0].T.astype(BF16)

    def b2d(wb):
        return wb[1].reshape(1, -1)

    h = _embed(x, t(p['W11']), b2d(p['W11']),
               p['ln1'][0].reshape(1, -1), p['ln1'][1].reshape(1, -1),
               t(p['W12']), b2d(p['W12']), bm=400)
    e = _embed(edge_attr, t(p['W21']), b2d(p['W21']),
               p['ln2'][0].reshape(1, -1), p['ln2'][1].reshape(1, -1),
               t(p['W22']), b2d(p['W22']), bm=640)

    npad = ((n + 127) // 128) * 128
    zeros_fc = jnp.zeros((npad, 128), F32)
    ne = src.shape[0]
    src3 = src.reshape(16, (ne // 16) // 80, 80)
    dst3 = dst.reshape(16, (ne // 16) // 80, 80)

    for lp in p['layers']:
        wtab = jnp.concatenate(
            [t(lp['A2']), t(lp['A3']), t(lp['B2']), t(lp['B3'])], axis=1)
        btab = jnp.concatenate(
            [b2d(lp['A2']), b2d(lp['A3']), b2d(lp['B2']), b2d(lp['B3'])],
            axis=1)
        a1h, tsrc, tdst = _nodeproj(h, lp['A1'][0].T, b2d(lp['A1']),
                                    wtab, btab, bm=400)
        gs = _gather_rows(tsrc, src)
        gd = _gather_rows(tdst, dst)
        e, mf, mb = _edgelayer(
            e, gs, gd, lp['B1'][0].T, b2d(lp['B1']),
            lp['ln_e'][0].reshape(1, -1), lp['ln_e'][1].reshape(1, -1),
            be=640)
        hs = _segsum2(mf, dst3, mb, src3, npad, zeros_fc)[:n]
        h = _nodeupd(h, a1h, hs,
                     lp['ln_h'][0].reshape(1, -1),
                     lp['ln_h'][1].reshape(1, -1), bm=400)

    w1, b1 = p['scorer1']
    wab = jnp.concatenate([w1[:, 0:512].T, w1[:, 512:1024].T],
                          axis=1).astype(BF16)
    ha, hb2 = _scoreproj(h, wab, bm=400)
    has = _gather_rows(ha, src)
    hbd = _gather_rows(hb2, dst)
    w2, b2 = p['scorer2']
    w2p = jnp.zeros((HID, 128), F32).at[:, 0].set(w2[0])
    b2p = jnp.broadcast_to(b2.reshape(1, 1), (1, 128)).astype(F32)
    sc = _scorefinal(has, hbd, e, w1[:, 1024:1536].T,
                     b1.reshape(1, -1), w2p, b2p, be=640)
    return sc[:, 0:1]
